# baseline (speedup 1.0000x reference)
import jax
import jax.numpy as jnp
from jax import lax
from jax.experimental import pallas as pl
from jax.experimental.pallas import tpu as pltpu
from jax.experimental.pallas import tpu_sc as plsc

N_FIELDS = 26
VOCAB = 100000
EMBED_DIM = 32
BATCH = 16384

NC, NS = 2, 16
NW = NC * NS
BPW = BATCH // NW
CHUNK = 16
NCHUNK = BPW // CHUNK
RPC = CHUNK * N_FIELDS
GROUP = 104
GPC = RPC // GROUP
WIDE = 128
HALF = 16


def _emb_body(tab_hbm, meta_hbm, out_hbm, meta_v, rows_v, acc_v, sem0, sem1):
    wid = lax.axis_index("s") * NC + lax.axis_index("c")
    sems = (sem0, sem1)

    def issue(c):
        buf = c % 2
        pltpu.sync_copy(meta_hbm.at[wid * NCHUNK + c], meta_v.at[buf])
        descs = []
        for j in range(GPC):
            descs.append(pltpu.async_copy(
                tab_hbm.at[meta_v.at[buf, j, pl.ds(0, GROUP)]],
                rows_v.at[buf, pl.ds(j * GROUP, GROUP)],
                sems[buf],
            ))
        return descs

    def compute(c):
        buf = c % 2
        arow = (c % 2) * 4

        def body(b, carry):
            r0 = b * N_FIELDS
            cv0 = meta_v[buf, 4 + b // 4, pl.ds((b % 4) * 32, HALF)]
            cv1 = meta_v[buf, 4 + b // 4, pl.ds((b % 4) * 32 + HALF, HALF)]
            a0 = None
            a1 = None
            for f in range(N_FIELDS):
                col = cv0[f] if f < HALF else cv1[f - HALF]
                v0 = rows_v[buf, r0 + f, pl.ds(col, HALF)]
                v1 = rows_v[buf, r0 + f, pl.ds(col + HALF, HALF)]
                a0 = v0 if a0 is None else a0 + v0
                a1 = v1 if a1 is None else a1 + v1
            acc_v[arow + b // 4, pl.ds((b % 4) * 32, HALF)] = a0
            acc_v[arow + b // 4, pl.ds((b % 4) * 32 + HALF, HALF)] = a1
            return carry

        lax.fori_loop(0, CHUNK, body, 0)
        if c % 2 == 1:
            pltpu.sync_copy(acc_v, out_hbm.at[wid * (NCHUNK // 2) + c // 2])

    descs = issue(0)
    for c in range(NCHUNK):
        next_descs = issue(c + 1) if c + 1 < NCHUNK else None
        for d in descs:
            d.wait()
        compute(c)
        descs = next_descs


def kernel(x_categorical, tables):
    offs = jnp.arange(N_FIELDS, dtype=jnp.int32) * VOCAB
    flat = x_categorical + offs[None, :]
    gid = (flat >> 2).reshape(NW * NCHUNK, GPC, GROUP)
    gid = jnp.pad(gid, ((0, 0), (0, 0), (0, WIDE - GROUP)))
    col = ((flat & 3) << 5).reshape(NW * NCHUNK, CHUNK, N_FIELDS)
    col = jnp.pad(col, ((0, 0), (0, 0), (0, 32 - N_FIELDS)))
    col = col.reshape(NW * NCHUNK, GPC, WIDE)
    meta = jnp.concatenate([gid, col], axis=1)

    tab = tables.reshape(N_FIELDS * VOCAB // 4, WIDE)

    run = pl.kernel(
        _emb_body,
        out_type=jax.ShapeDtypeStruct((NW * NCHUNK // 2, 8, WIDE),
                                      jnp.float32),
        mesh=plsc.VectorSubcoreMesh(
            core_axis_name="c", subcore_axis_name="s",
            num_cores=NC, num_subcores=NS),
        scratch_types=[
            pltpu.VMEM((2, 8, WIDE), jnp.int32),
            pltpu.VMEM((2, RPC, WIDE), jnp.float32),
            pltpu.VMEM((8, WIDE), jnp.float32),
            pltpu.SemaphoreType.DMA,
            pltpu.SemaphoreType.DMA,
        ],
        compiler_params=pltpu.CompilerParams(use_tc_tiling_on_sc=True),
    )
    return run(tab, meta).reshape(BATCH, EMBED_DIM)


# speedup vs baseline: 1.3944x; 1.3944x over previous
import jax
import jax.numpy as jnp
from jax import lax
from jax.experimental import pallas as pl
from jax.experimental.pallas import tpu as pltpu
from jax.experimental.pallas import tpu_sc as plsc

N_FIELDS = 26
VOCAB = 100000
EMBED_DIM = 32
BATCH = 16384

VB = 4096
TBLK = 25
TROWS = TBLK * VB // 4
WROWS = N_FIELDS * TROWS

NC, NS = 2, 16
NW = NC * NS
BPW = BATCH // NW
CHUNK = 16
NCHUNK = BPW // CHUNK
RPC = CHUNK * N_FIELDS
GROUP = 104
GPC = RPC // GROUP
WIDE = 128
HALF = 16


def _transpose_body(x_ref, w_ref):
    x = x_ref[0]
    y = jnp.swapaxes(x, 0, 1)
    q = VB // 4
    for j in range(4):
        w_ref[:, j * EMBED_DIM:(j + 1) * EMBED_DIM] = y[j * q:(j + 1) * q]


def _emb_body(tab_hbm, meta_hbm, out_hbm, meta_v, rows_v, acc_v, sem0, sem1):
    wid = lax.axis_index("s") * NC + lax.axis_index("c")
    sems = (sem0, sem1)

    def issue(c):
        buf = c % 2
        pltpu.sync_copy(meta_hbm.at[wid * NCHUNK + c], meta_v.at[buf])
        descs = []
        for j in range(GPC):
            descs.append(pltpu.async_copy(
                tab_hbm.at[meta_v.at[buf, j, pl.ds(0, GROUP)]],
                rows_v.at[buf, pl.ds(j * GROUP, GROUP)],
                sems[buf],
            ))
        return descs

    def compute(c):
        buf = c % 2
        arow = (c % 2) * 4

        def body(b, carry):
            r0 = b * N_FIELDS
            cv0 = meta_v[buf, 4 + b // 4, pl.ds((b % 4) * 32, HALF)]
            cv1 = meta_v[buf, 4 + b // 4, pl.ds((b % 4) * 32 + HALF, HALF)]
            a0 = None
            a1 = None
            for f in range(N_FIELDS):
                col = cv0[f] if f < HALF else cv1[f - HALF]
                v0 = rows_v[buf, r0 + f, pl.ds(col, HALF)]
                v1 = rows_v[buf, r0 + f, pl.ds(col + HALF, HALF)]
                a0 = v0 if a0 is None else a0 + v0
                a1 = v1 if a1 is None else a1 + v1
            acc_v[arow + b // 4, pl.ds((b % 4) * 32, HALF)] = a0
            acc_v[arow + b // 4, pl.ds((b % 4) * 32 + HALF, HALF)] = a1
            return carry

        lax.fori_loop(0, CHUNK, body, 0)
        if c % 2 == 1:
            pltpu.sync_copy(acc_v, out_hbm.at[wid * (NCHUNK // 2) + c // 2])

    descs = issue(0)
    for c in range(NCHUNK):
        next_descs = issue(c + 1) if c + 1 < NCHUNK else None
        for d in descs:
            d.wait()
        compute(c)
        descs = next_descs


def kernel(x_categorical, tables):
    tab_t = jnp.transpose(tables, (0, 2, 1))
    w = pl.pallas_call(
        _transpose_body,
        grid=(N_FIELDS, TBLK),
        in_specs=[pl.BlockSpec((1, EMBED_DIM, VB), lambda t, j: (t, 0, j))],
        out_specs=pl.BlockSpec((VB // 4, WIDE), lambda t, j: (t * TBLK + j, 0)),
        out_shape=jax.ShapeDtypeStruct((WROWS, WIDE), jnp.float32),
    )(tab_t)

    offs = jnp.arange(N_FIELDS, dtype=jnp.int32) * TROWS
    x = x_categorical
    gid = ((x >> 12) << 10) + (x & 1023) + offs[None, :]
    gid = gid.reshape(NW * NCHUNK, GPC, GROUP)
    gid = jnp.pad(gid, ((0, 0), (0, 0), (0, WIDE - GROUP)))
    col = (((x >> 10) & 3) << 5).reshape(NW * NCHUNK, CHUNK, N_FIELDS)
    col = jnp.pad(col, ((0, 0), (0, 0), (0, 32 - N_FIELDS)))
    col = col.reshape(NW * NCHUNK, GPC, WIDE)
    meta = jnp.concatenate([gid, col], axis=1)

    run = pl.kernel(
        _emb_body,
        out_type=jax.ShapeDtypeStruct((NW * NCHUNK // 2, 8, WIDE),
                                      jnp.float32),
        mesh=plsc.VectorSubcoreMesh(
            core_axis_name="c", subcore_axis_name="s",
            num_cores=NC, num_subcores=NS),
        scratch_types=[
            pltpu.VMEM((2, 8, WIDE), jnp.int32),
            pltpu.VMEM((2, RPC, WIDE), jnp.float32),
            pltpu.VMEM((8, WIDE), jnp.float32),
            pltpu.SemaphoreType.DMA,
            pltpu.SemaphoreType.DMA,
        ],
        compiler_params=pltpu.CompilerParams(use_tc_tiling_on_sc=True),
    )
    return run(w, meta).reshape(BATCH, EMBED_DIM)


# speedup vs baseline: 2.6791x; 1.9213x over previous
import jax
import jax.numpy as jnp
from jax import lax
from jax.experimental import pallas as pl
from jax.experimental.pallas import tpu as pltpu
from jax.experimental.pallas import tpu_sc as plsc

N_FIELDS = 26
VOCAB = 100000
EMBED_DIM = 32
BATCH = 16384

NC, NS = 2, 16
NW = NC * NS
LANES = 16
ICHUNK = 4096
NI = BATCH // ICHUNK


def _emb_body(tab_hbm, xt_hbm, out_hbm, plane_v, idx_v, acc_v):
    dw = lax.axis_index("s") * NC + lax.axis_index("c")

    def do_field(f, first):
        pltpu.sync_copy(tab_hbm.at[f * EMBED_DIM + dw], plane_v)

        def chunk_body(c, carry):
            pltpu.sync_copy(xt_hbm.at[f, pl.ds(c * ICHUNK, ICHUNK)], idx_v)

            def gather_body(i, carry2):
                s = c * ICHUNK + i * LANES
                g = plsc.load_gather(plane_v, [idx_v[pl.ds(i * LANES, LANES)]])
                if first:
                    acc_v[pl.ds(s, LANES)] = g
                else:
                    acc_v[pl.ds(s, LANES)] += g
                return carry2

            lax.fori_loop(0, ICHUNK // LANES, gather_body, 0, unroll=8)
            return carry

        lax.fori_loop(0, NI, chunk_body, 0)

    do_field(0, True)

    def field_body(f, carry):
        do_field(f, False)
        return carry

    lax.fori_loop(1, N_FIELDS, field_body, 0)
    pltpu.sync_copy(acc_v, out_hbm.at[dw])


def kernel(x_categorical, tables):
    tab = jnp.transpose(tables, (0, 2, 1)).reshape(N_FIELDS * EMBED_DIM, VOCAB)
    xt = jnp.transpose(x_categorical)

    run = pl.kernel(
        _emb_body,
        out_type=jax.ShapeDtypeStruct((EMBED_DIM, BATCH), jnp.float32),
        mesh=plsc.VectorSubcoreMesh(
            core_axis_name="c", subcore_axis_name="s",
            num_cores=NC, num_subcores=NS),
        scratch_types=[
            pltpu.VMEM((VOCAB,), jnp.float32),
            pltpu.VMEM((ICHUNK,), jnp.int32),
            pltpu.VMEM((BATCH,), jnp.float32),
        ],
        compiler_params=pltpu.CompilerParams(needs_layout_passes=False),
    )
    return jnp.transpose(run(tab, xt))


# speedup vs baseline: 4.7121x; 1.7588x over previous
import jax
import jax.numpy as jnp
from jax import lax
from jax.experimental import pallas as pl
from jax.experimental.pallas import tpu as pltpu
from jax.experimental.pallas import tpu_sc as plsc

N_FIELDS = 26
VOCAB = 100000
EMBED_DIM = 32
BATCH = 16384

NC, NS = 2, 16
NW = NC * NS
LANES = 16
ICHUNK = 4096
NI = BATCH // ICHUNK


def _emb_body(tab_hbm, xt_hbm, out_hbm, plane_v, idx_v, acc_v):
    dw = lax.axis_index("s") * NC + lax.axis_index("c")

    def do_field(f, first):
        pltpu.sync_copy(tab_hbm.at[f * EMBED_DIM + dw], plane_v)

        def chunk_body(c, carry):
            pltpu.sync_copy(xt_hbm.at[f, pl.ds(c * ICHUNK, ICHUNK)], idx_v)

            @plsc.parallel_loop(0, ICHUNK // LANES, unroll=8)
            def gather_body(i):
                s = c * ICHUNK + i * LANES
                g = plsc.load_gather(plane_v, [idx_v[pl.ds(i * LANES, LANES)]])
                if first:
                    acc_v[pl.ds(s, LANES)] = g
                else:
                    plsc.addupdate(acc_v.at[pl.ds(s, LANES)], g)
            return carry

        lax.fori_loop(0, NI, chunk_body, 0)

    do_field(0, True)

    def field_body(f, carry):
        do_field(f, False)
        return carry

    lax.fori_loop(1, N_FIELDS, field_body, 0)
    pltpu.sync_copy(acc_v, out_hbm.at[dw])


def kernel(x_categorical, tables):
    tab = jnp.transpose(tables, (0, 2, 1)).reshape(N_FIELDS * EMBED_DIM, VOCAB)
    xt = jnp.transpose(x_categorical)

    run = pl.kernel(
        _emb_body,
        out_type=jax.ShapeDtypeStruct((EMBED_DIM, BATCH), jnp.float32),
        mesh=plsc.VectorSubcoreMesh(
            core_axis_name="c", subcore_axis_name="s",
            num_cores=NC, num_subcores=NS),
        scratch_types=[
            pltpu.VMEM((VOCAB,), jnp.float32),
            pltpu.VMEM((ICHUNK,), jnp.int32),
            pltpu.VMEM((BATCH,), jnp.float32),
        ],
        compiler_params=pltpu.CompilerParams(needs_layout_passes=False),
    )
    return jnp.transpose(run(tab, xt))


# speedup vs baseline: 5.7198x; 1.2139x over previous
import jax
import jax.numpy as jnp
from jax import lax
from jax.experimental import pallas as pl
from jax.experimental.pallas import tpu as pltpu
from jax.experimental.pallas import tpu_sc as plsc

N_FIELDS = 26
VOCAB = 100000
EMBED_DIM = 32
BATCH = 16384

NC, NS = 2, 16
NW = NC * NS
LANES = 16
ICHUNK = 4096
NI = BATCH // ICHUNK
NIDX = N_FIELDS * BATCH


def _emb_body(tab_hbm, xt_hbm, out_hbm, plane_v, idx_v, acc_v, sem_i):
    sid = lax.axis_index("s")
    dw = sid * NC + lax.axis_index("c")

    def do_field(f, first):
        pltpu.sync_copy(tab_hbm.at[f * EMBED_DIM + dw], plane_v)
        descs = [pltpu.async_copy(xt_hbm.at[pl.ds(f * BATCH, ICHUNK)],
                                  idx_v.at[0], sem_i)]
        for c in range(NI):
            if c + 1 < NI:
                descs.append(pltpu.async_copy(
                    xt_hbm.at[pl.ds(f * BATCH + (c + 1) * ICHUNK, ICHUNK)],
                    idx_v.at[(c + 1) % 2], sem_i))
            descs[c].wait()

            @plsc.parallel_loop(0, ICHUNK // LANES, unroll=8)
            def gather_body(i):
                s = c * ICHUNK + i * LANES
                g = plsc.load_gather(
                    plane_v, [idx_v[c % 2, pl.ds(i * LANES, LANES)]])
                if first:
                    acc_v[pl.ds(s, LANES)] = g
                else:
                    plsc.addupdate(acc_v.at[pl.ds(s, LANES)], g)

    do_field(0, True)

    def field_body(f, carry):
        do_field(f, False)
        return carry

    lax.fori_loop(1, N_FIELDS, field_body, 0)
    pltpu.sync_copy(acc_v, out_hbm.at[dw])


def kernel(x_categorical, tables):
    tab = jnp.transpose(tables, (0, 2, 1)).reshape(N_FIELDS * EMBED_DIM, VOCAB)
    xt = jnp.transpose(x_categorical).reshape(NIDX)

    run = pl.kernel(
        _emb_body,
        out_type=jax.ShapeDtypeStruct((EMBED_DIM, BATCH), jnp.float32),
        mesh=plsc.VectorSubcoreMesh(
            core_axis_name="c", subcore_axis_name="s",
            num_cores=NC, num_subcores=NS),
        scratch_types=[
            pltpu.VMEM((VOCAB,), jnp.float32),
            pltpu.VMEM((2, ICHUNK), jnp.int32),
            pltpu.VMEM((BATCH,), jnp.float32),
            pltpu.SemaphoreType.DMA,
        ],
        compiler_params=pltpu.CompilerParams(needs_layout_passes=False),
    )
    return jnp.transpose(run(tab, xt))


# speedup vs baseline: 6.1665x; 1.0781x over previous
import jax
import jax.numpy as jnp
from jax import lax
from jax.experimental import pallas as pl
from jax.experimental.pallas import tpu as pltpu
from jax.experimental.pallas import tpu_sc as plsc

N_FIELDS = 26
VOCAB = 100000
EMBED_DIM = 32
BATCH = 16384

NC, NS = 2, 16
NW = NC * NS
LANES = 16
ICHUNK = 2048
NI = BATCH // ICHUNK
NIDX = N_FIELDS * BATCH
RFIELDS = 9
ROUNDS = ((0, 9), (9, 9), (18, 8))


def _emb_body(tab_hbm, xt_hbm, out_hbm, plane_v, idx_v, acc_v, stage_sp,
              sem_i):
    sid = lax.axis_index("s")
    dw = sid * NC + lax.axis_index("c")

    def do_field(base_f, fr, first):
        pltpu.sync_copy(tab_hbm.at[(base_f * EMBED_DIM + fr * EMBED_DIM) + dw],
                        plane_v)
        descs = [pltpu.async_copy(stage_sp.at[pl.ds(fr * BATCH, ICHUNK)],
                                  idx_v.at[0], sem_i)]
        for c in range(NI):
            if c + 1 < NI:
                descs.append(pltpu.async_copy(
                    stage_sp.at[pl.ds(fr * BATCH + (c + 1) * ICHUNK, ICHUNK)],
                    idx_v.at[(c + 1) % 2], sem_i))
            descs[c].wait()

            @plsc.parallel_loop(0, ICHUNK // LANES, unroll=8)
            def gather_body(i):
                s = c * ICHUNK + i * LANES
                g = plsc.load_gather(
                    plane_v, [idx_v[c % 2, pl.ds(i * LANES, LANES)]])
                if first:
                    acc_v[pl.ds(s, LANES)] = g
                else:
                    plsc.addupdate(acc_v.at[pl.ds(s, LANES)], g)

    for base_f, nf in ROUNDS:
        if base_f > 0:
            plsc.subcore_barrier()
        share = nf * BATCH // NS
        pltpu.sync_copy(
            xt_hbm.at[pl.ds(base_f * BATCH + sid * share, share)],
            stage_sp.at[pl.ds(sid * share, share)])
        plsc.subcore_barrier()

        if base_f == 0:
            do_field(0, 0, True)

            def field_body(fr, carry):
                do_field(0, fr, False)
                return carry

            lax.fori_loop(1, nf, field_body, 0)
        else:
            def field_body(fr, carry):
                do_field(base_f, fr, False)
                return carry

            lax.fori_loop(0, nf, field_body, 0)

    pltpu.sync_copy(acc_v, out_hbm.at[dw])


def kernel(x_categorical, tables):
    tab = jnp.transpose(tables, (0, 2, 1)).reshape(N_FIELDS * EMBED_DIM, VOCAB)
    xt = jnp.transpose(x_categorical).reshape(NIDX)

    run = pl.kernel(
        _emb_body,
        out_type=jax.ShapeDtypeStruct((EMBED_DIM, BATCH), jnp.float32),
        mesh=plsc.VectorSubcoreMesh(
            core_axis_name="c", subcore_axis_name="s",
            num_cores=NC, num_subcores=NS),
        scratch_types=[
            pltpu.VMEM((VOCAB,), jnp.float32),
            pltpu.VMEM((2, ICHUNK), jnp.int32),
            pltpu.VMEM((BATCH,), jnp.float32),
            pltpu.VMEM_SHARED((RFIELDS * BATCH,), jnp.int32),
            pltpu.SemaphoreType.DMA,
        ],
        compiler_params=pltpu.CompilerParams(needs_layout_passes=False),
    )
    return jnp.transpose(run(tab, xt))
